# Initial kernel scaffold; baseline (speedup 1.0000x reference)
import functools

import jax
import jax.numpy as jnp
from jax import lax
from jax.experimental import pallas as pl
from jax.experimental.pallas import tpu as pltpu
from jax.experimental.pallas import tpu_sc as plsc

_N = 10000
_D = 256
_H = 128
_E = 160000
_NT = 16
_NC = 2
_CHUNK = 128
_GROUPS = _NC * _NT
_CPG = 40
_EP = _GROUPS * _CPG * _CHUNK
_ACC_ROWS = 10240
_RT = _ACC_ROWS // _NT
_PAD_ROW = _N
_R = 1000

_mesh = plsc.VectorSubcoreMesh(core_axis_name="c", subcore_axis_name="s")


@functools.partial(
    pl.kernel,
    out_type=jax.ShapeDtypeStruct((_ACC_ROWS, 16), jnp.float32),
    mesh=_mesh,
    scratch_types=[
        pltpu.VMEM((2 * _CPG, _CHUNK), jnp.int32),
        pltpu.VMEM((_CHUNK, 16), jnp.float32),
        pltpu.VMEM_SHARED((_ACC_ROWS, 16), jnp.float32),
    ],
)
def _sc_degree(dst_hbm, ones_hbm, z16_hbm, deg_hbm, dstc, ones_v, acc):
    c = lax.axis_index("c")
    s = lax.axis_index("s")

    @pl.when(c == 0)
    def _():
        pltpu.sync_copy(z16_hbm, acc.at[pl.ds(s * _RT, _RT)])
        pltpu.sync_copy(ones_hbm, ones_v)
        pltpu.sync_copy(dst_hbm.at[2 * s], dstc.at[pl.ds(0, _CPG)])
        pltpu.sync_copy(dst_hbm.at[2 * s + 1], dstc.at[pl.ds(_CPG, _CPG)])
        plsc.subcore_barrier()

        def body(j, carry):
            pltpu.sync_copy(ones_v, acc.at[dstc.at[j]], add=True)
            return carry

        lax.fori_loop(0, 2 * _CPG, body, 0)
        plsc.subcore_barrier()
        pltpu.sync_copy(acc.at[pl.ds(s * _RT, _RT)],
                        deg_hbm.at[pl.ds(s * _RT, _RT)])


def _conv_half(tbl, src_hbm, dst_hbm, zrow_hbm, out_half, srcc, dstc, rbuf, acc, s):
    pltpu.sync_copy(zrow_hbm, acc.at[pl.ds(s * _RT, _RT)])
    pltpu.sync_copy(src_hbm.at[2 * s], srcc.at[pl.ds(0, _CPG)])
    pltpu.sync_copy(src_hbm.at[2 * s + 1], srcc.at[pl.ds(_CPG, _CPG)])
    pltpu.sync_copy(dst_hbm.at[2 * s], dstc.at[pl.ds(0, _CPG)])
    pltpu.sync_copy(dst_hbm.at[2 * s + 1], dstc.at[pl.ds(_CPG, _CPG)])
    plsc.subcore_barrier()

    def body(j, carry):
        pltpu.sync_copy(tbl.at[srcc.at[j]], rbuf)
        pltpu.sync_copy(rbuf, acc.at[dstc.at[j]], add=True)
        return carry

    lax.fori_loop(0, 2 * _CPG, body, 0)
    plsc.subcore_barrier()
    pltpu.sync_copy(acc.at[pl.ds(s * _RT, _RT)], out_half.at[pl.ds(s * _RT, _RT)])


@functools.partial(
    pl.kernel,
    out_type=jax.ShapeDtypeStruct((_NC, _ACC_ROWS, _H), jnp.float32),
    mesh=_mesh,
    scratch_types=[
        pltpu.VMEM((2 * _CPG, _CHUNK), jnp.int32),
        pltpu.VMEM((2 * _CPG, _CHUNK), jnp.int32),
        pltpu.VMEM((_CHUNK, _H), jnp.float32),
        pltpu.VMEM_SHARED((_ACC_ROWS, _H), jnp.float32),
    ],
)
def _sc_conv(hs0, hs1, src_hbm, dst_hbm, zrow_hbm, out_hbm, srcc, dstc, rbuf, acc):
    c = lax.axis_index("c")
    s = lax.axis_index("s")

    @pl.when(c == 0)
    def _():
        _conv_half(hs0, src_hbm, dst_hbm, zrow_hbm, out_hbm.at[0],
                   srcc, dstc, rbuf, acc, s)

    @pl.when(c == 1)
    def _():
        _conv_half(hs1, src_hbm, dst_hbm, zrow_hbm, out_hbm.at[1],
                   srcc, dstc, rbuf, acc, s)


def _tc1_body(x_ref, w1_ref, deg_ref, hs0_ref, hs1_ref, dinv_ref):
    h = jnp.dot(x_ref[...], w1_ref[...], preferred_element_type=jnp.float32)
    dinv = lax.rsqrt(deg_ref[...] + 1.0)
    hs = h * dinv
    hs0_ref[...] = hs[:, :_H]
    hs1_ref[...] = hs[:, _H:]
    dinv_ref[...] = jnp.broadcast_to(dinv, (_R, _H))


def _tc2_body(a_ref, hs0_ref, hs1_ref, dinv_ref, b1_ref, wl_ref, bl_ref,
              w2_ref, o0_ref, o1_ref):
    dinv = dinv_ref[...]
    g0 = jnp.maximum(dinv * (a_ref[0] + hs0_ref[...]) + b1_ref[0, :_H], 0.0)
    g1 = jnp.maximum(dinv * (a_ref[1] + hs1_ref[...]) + b1_ref[0, _H:], 0.0)
    g = jnp.concatenate([g0, g1], axis=1)
    z = jnp.maximum(
        jnp.dot(g, wl_ref[...], preferred_element_type=jnp.float32) + bl_ref[...],
        0.0)
    h2 = jnp.dot(z, w2_ref[...], preferred_element_type=jnp.float32)
    hs2 = h2 * dinv
    o0_ref[...] = hs2[:, :_H]
    o1_ref[...] = hs2[:, _H:]


def _tc3_body(a_ref, hs0_ref, hs1_ref, dinv_ref, b2_ref, out_ref):
    dinv = dinv_ref[...]
    o0 = dinv * (a_ref[0] + hs0_ref[...]) + b2_ref[0, :_H]
    o1 = dinv * (a_ref[1] + hs1_ref[...]) + b2_ref[0, _H:]
    out_ref[...] = jnp.concatenate([o0, o1], axis=1)


_row = pl.BlockSpec((_R, _D), lambda i: (i, 0))
_half = pl.BlockSpec((_R, _H), lambda i: (i, 0))
_whole = pl.BlockSpec((_D, _D), lambda i: (0, 0))
_bias = pl.BlockSpec((1, _D), lambda i: (0, 0))
_acc_spec = pl.BlockSpec((_NC, _R, _H), lambda i: (0, i, 0))

_tc1 = pl.pallas_call(
    _tc1_body,
    grid=(_N // _R,),
    in_specs=[_row, _whole, pl.BlockSpec((_R, 1), lambda i: (i, 0))],
    out_specs=[_half, _half, _half],
    out_shape=[jax.ShapeDtypeStruct((_N, _H), jnp.float32)] * 3,
)

_tc2 = pl.pallas_call(
    _tc2_body,
    grid=(_N // _R,),
    in_specs=[_acc_spec, _half, _half, _half, _bias, _whole, _bias, _whole],
    out_specs=[_half, _half],
    out_shape=[jax.ShapeDtypeStruct((_N, _H), jnp.float32)] * 2,
)

_tc3 = pl.pallas_call(
    _tc3_body,
    grid=(_N // _R,),
    in_specs=[_acc_spec, _half, _half, _half, _bias],
    out_specs=_row,
    out_shape=jax.ShapeDtypeStruct((_N, _D), jnp.float32),
)


def kernel(x, adj, W1, b1, Wl, bl, W2, b2):
    src = adj[0]
    dst = adj[1]
    pad = _EP - _E
    srcp = jnp.concatenate(
        [src, jnp.zeros((pad,), jnp.int32)]).reshape(_GROUPS, _CPG, _CHUNK)
    dstp = jnp.concatenate(
        [dst, jnp.full((pad,), _PAD_ROW, jnp.int32)]).reshape(_GROUPS, _CPG, _CHUNK)
    ones16 = jnp.ones((_CHUNK, 16), jnp.float32)
    z16 = jnp.zeros((_RT, 16), jnp.float32)
    zrow = jnp.zeros((_RT, _H), jnp.float32)

    degf = _sc_degree(dstp, ones16, z16)
    deg = degf[:_N, :1]

    hs0, hs1, dinv = _tc1(x, W1, deg)

    acc1 = _sc_conv(hs0, hs1, srcp, dstp, zrow)
    hs20, hs21 = _tc2(acc1[:, :_N, :], hs0, hs1, dinv,
                      b1.reshape(1, _D), Wl, bl.reshape(1, _D), W2)

    acc2 = _sc_conv(hs20, hs21, srcp, dstp, zrow)
    return _tc3(acc2[:, :_N, :], hs20, hs21, dinv, b2.reshape(1, _D))


# speedup vs baseline: 6.7842x; 6.7842x over previous
import functools

import jax
import jax.numpy as jnp
from jax import lax
from jax.experimental import pallas as pl
from jax.experimental.pallas import tpu as pltpu
from jax.experimental.pallas import tpu_sc as plsc

_N = 10000
_D = 256
_H = 128
_E = 160000
_NT = 16
_NC = 2
_CHUNK = 128
_GROUPS = _NC * _NT
_CPG = 40
_EP = _GROUPS * _CPG * _CHUNK
_ACC_ROWS = 10240
_RT = _ACC_ROWS // _NT
_PAD_ROW = _N
_R = 1000


def _sc_degree_body(dst_hbm, ones_hbm, zrow_hbm, deg_hbm, dstc, ones_v, acc):
    c = lax.axis_index("c")
    s = lax.axis_index("s")
    pltpu.sync_copy(zrow_hbm, acc.at[pl.ds(s * _RT, _RT)])
    pltpu.sync_copy(ones_hbm, ones_v)
    pltpu.sync_copy(dst_hbm.at[c * _NT + s], dstc)
    plsc.subcore_barrier()

    def body(j, carry):
        pltpu.sync_copy(ones_v, acc.at[dstc.at[j]], add=True)
        return carry

    lax.fori_loop(0, _CPG, body, 0)
    plsc.subcore_barrier()

    @pl.when(c == 0)
    def _():
        pltpu.sync_copy(acc.at[pl.ds(s * _RT, _RT)],
                        deg_hbm.at[0, pl.ds(s * _RT, _RT)])

    @pl.when(c == 1)
    def _():
        pltpu.sync_copy(acc.at[pl.ds(s * _RT, _RT)],
                        deg_hbm.at[1, pl.ds(s * _RT, _RT)])


def _conv_half(tbl, src_hbm, dst_hbm, zrow_hbm, out_half, srcc, dstc, rbuf, acc, s):
    pltpu.sync_copy(zrow_hbm, acc.at[pl.ds(s * _RT, _RT)])
    pltpu.sync_copy(src_hbm.at[2 * s], srcc.at[pl.ds(0, _CPG)])
    pltpu.sync_copy(src_hbm.at[2 * s + 1], srcc.at[pl.ds(_CPG, _CPG)])
    pltpu.sync_copy(dst_hbm.at[2 * s], dstc.at[pl.ds(0, _CPG)])
    pltpu.sync_copy(dst_hbm.at[2 * s + 1], dstc.at[pl.ds(_CPG, _CPG)])
    plsc.subcore_barrier()

    def body(j, carry):
        pltpu.sync_copy(tbl.at[srcc.at[j]], rbuf)
        pltpu.sync_copy(rbuf, acc.at[dstc.at[j]], add=True)
        return carry

    lax.fori_loop(0, 2 * _CPG, body, 0)
    plsc.subcore_barrier()
    pltpu.sync_copy(acc.at[pl.ds(s * _RT, _RT)], out_half.at[pl.ds(s * _RT, _RT)])


def _sc_conv_body(hs0, hs1, src_hbm, dst_hbm, zrow_hbm, out_hbm, srcc, dstc, rbuf, acc):
    c = lax.axis_index("c")
    s = lax.axis_index("s")

    @pl.when(c == 0)
    def _():
        _conv_half(hs0, src_hbm, dst_hbm, zrow_hbm, out_hbm.at[0],
                   srcc, dstc, rbuf, acc, s)

    @pl.when(c == 1)
    def _():
        _conv_half(hs1, src_hbm, dst_hbm, zrow_hbm, out_hbm.at[1],
                   srcc, dstc, rbuf, acc, s)


@functools.cache
def _sc_kernels():
    mesh = plsc.VectorSubcoreMesh(
        core_axis_name="c", subcore_axis_name="s",
        num_cores=_NC, num_subcores=_NT)
    sc_degree = pl.kernel(
        _sc_degree_body,
        out_type=jax.ShapeDtypeStruct((_NC, _ACC_ROWS, _H), jnp.float32),
        mesh=mesh,
        scratch_types=[
            pltpu.VMEM((_CPG, _CHUNK), jnp.int32),
            pltpu.VMEM((_CHUNK, _H), jnp.float32),
            pltpu.VMEM_SHARED((_ACC_ROWS, _H), jnp.float32),
        ],
    )
    sc_conv = pl.kernel(
        _sc_conv_body,
        out_type=jax.ShapeDtypeStruct((_NC, _ACC_ROWS, _H), jnp.float32),
        mesh=mesh,
        scratch_types=[
            pltpu.VMEM((2 * _CPG, _CHUNK), jnp.int32),
            pltpu.VMEM((2 * _CPG, _CHUNK), jnp.int32),
            pltpu.VMEM((_CHUNK, _H), jnp.float32),
            pltpu.VMEM_SHARED((_ACC_ROWS, _H), jnp.float32),
        ],
    )
    return sc_degree, sc_conv


def _tc1_body(x_ref, w1_ref, deg_ref, hs0_ref, hs1_ref, dinv_ref):
    h = jnp.dot(x_ref[...], w1_ref[...], preferred_element_type=jnp.float32)
    dinv = lax.rsqrt(deg_ref[0] + deg_ref[1] + 1.0)
    hs = h * dinv
    hs0_ref[...] = hs[:, :_H]
    hs1_ref[...] = hs[:, _H:]
    dinv_ref[...] = jnp.broadcast_to(dinv, (_R, _H))


def _tc2_body(a_ref, hs0_ref, hs1_ref, dinv_ref, b1_ref, wl_ref, bl_ref,
              w2_ref, o0_ref, o1_ref):
    dinv = dinv_ref[...]
    g0 = jnp.maximum(dinv * (a_ref[0] + hs0_ref[...]) + b1_ref[0, :_H], 0.0)
    g1 = jnp.maximum(dinv * (a_ref[1] + hs1_ref[...]) + b1_ref[0, _H:], 0.0)
    g = jnp.concatenate([g0, g1], axis=1)
    z = jnp.maximum(
        jnp.dot(g, wl_ref[...], preferred_element_type=jnp.float32) + bl_ref[...],
        0.0)
    h2 = jnp.dot(z, w2_ref[...], preferred_element_type=jnp.float32)
    o0_ref[...] = h2[:, :_H] * dinv
    o1_ref[...] = h2[:, _H:] * dinv


def _tc3_body(a_ref, hs0_ref, hs1_ref, dinv_ref, b2_ref, out_ref):
    dinv = dinv_ref[...]
    o0 = dinv * (a_ref[0] + hs0_ref[...]) + b2_ref[0, :_H]
    o1 = dinv * (a_ref[1] + hs1_ref[...]) + b2_ref[0, _H:]
    out_ref[...] = jnp.concatenate([o0, o1], axis=1)


_row = pl.BlockSpec((_R, _D), lambda i: (i, 0))
_half = pl.BlockSpec((_R, _H), lambda i: (i, 0))
_whole = pl.BlockSpec((_D, _D), lambda i: (0, 0))
_bias = pl.BlockSpec((1, _D), lambda i: (0, 0))
_acc_spec = pl.BlockSpec((_NC, _R, _H), lambda i: (0, i, 0))

_tc1 = pl.pallas_call(
    _tc1_body,
    grid=(_N // _R,),
    in_specs=[_row, _whole, pl.BlockSpec((_NC, _R, 1), lambda i: (0, i, 0))],
    out_specs=[_half, _half, _half],
    out_shape=[jax.ShapeDtypeStruct((_N, _H), jnp.float32)] * 3,
)

_tc2 = pl.pallas_call(
    _tc2_body,
    grid=(_N // _R,),
    in_specs=[_acc_spec, _half, _half, _half, _bias, _whole, _bias, _whole],
    out_specs=[_half, _half],
    out_shape=[jax.ShapeDtypeStruct((_N, _H), jnp.float32)] * 2,
)

_tc3 = pl.pallas_call(
    _tc3_body,
    grid=(_N // _R,),
    in_specs=[_acc_spec, _half, _half, _half, _bias],
    out_specs=_row,
    out_shape=jax.ShapeDtypeStruct((_N, _D), jnp.float32),
)


def kernel(x, adj, W1, b1, Wl, bl, W2, b2):
    src = adj[0]
    dst = adj[1]
    pad = _EP - _E
    srcp = jnp.concatenate(
        [src, jnp.zeros((pad,), jnp.int32)]).reshape(_GROUPS, _CPG, _CHUNK)
    dstp = jnp.concatenate(
        [dst, jnp.full((pad,), _PAD_ROW, jnp.int32)]).reshape(_GROUPS, _CPG, _CHUNK)
    ones128 = jnp.ones((_CHUNK, _H), jnp.float32)
    zrow = jnp.zeros((_RT, _H), jnp.float32)

    _sc_degree, _sc_conv = _sc_kernels()
    degf = _sc_degree(dstp, ones128, zrow)
    deg = degf[:, :_N, :1]

    hs0, hs1, dinv = _tc1(x, W1, deg)

    acc1 = _sc_conv(hs0, hs1, srcp, dstp, zrow)
    hs20, hs21 = _tc2(acc1[:, :_N, :], hs0, hs1, dinv,
                      b1.reshape(1, _D), Wl, bl.reshape(1, _D), W2)

    acc2 = _sc_conv(hs20, hs21, srcp, dstp, zrow)
    return _tc3(acc2[:, :_N, :], hs20, hs21, dinv, b2.reshape(1, _D))


# speedup vs baseline: 7.4773x; 1.1022x over previous
import functools

import jax
import jax.numpy as jnp
from jax import lax
from jax.experimental import pallas as pl
from jax.experimental.pallas import tpu as pltpu
from jax.experimental.pallas import tpu_sc as plsc

_N = 10000
_D = 256
_H = 128
_E = 160000
_NT = 16
_NC = 2
_CHUNK = 128
_GROUPS = _NC * _NT
_CPG = 40
_EP = _GROUPS * _CPG * _CHUNK
_ACC_ROWS = 10240
_RT = _ACC_ROWS // _NT
_PAD_ROW = _N
_R = 1000


def _sc_degree_body(dst_hbm, ones_hbm, zrow_hbm, deg_hbm, dstc, ones_v, acc):
    c = lax.axis_index("c")
    s = lax.axis_index("s")
    pltpu.sync_copy(zrow_hbm, acc.at[pl.ds(s * _RT, _RT)])
    pltpu.sync_copy(ones_hbm, ones_v)
    pltpu.sync_copy(dst_hbm.at[c * _NT + s], dstc)
    plsc.subcore_barrier()

    def body(j, carry):
        pltpu.sync_copy(ones_v, acc.at[dstc.at[j]], add=True)
        return carry

    lax.fori_loop(0, _CPG, body, 0)
    plsc.subcore_barrier()

    @pl.when(c == 0)
    def _():
        pltpu.sync_copy(acc.at[pl.ds(s * _RT, _RT)],
                        deg_hbm.at[0, pl.ds(s * _RT, _RT)])

    @pl.when(c == 1)
    def _():
        pltpu.sync_copy(acc.at[pl.ds(s * _RT, _RT)],
                        deg_hbm.at[1, pl.ds(s * _RT, _RT)])


def _conv_phase(tbl, src_g, dst_g, srcc, dstc, rbuf, acc, gsem0, gsem1):
    pltpu.sync_copy(src_g, srcc)
    pltpu.sync_copy(dst_g, dstc)
    pltpu.async_copy(tbl.at[srcc.at[0]], rbuf.at[0], gsem0)

    def body(t, carry):
        j = 2 * t
        pltpu.make_async_copy(tbl.at[srcc.at[j]], rbuf.at[0], gsem0).wait()
        pltpu.async_copy(tbl.at[srcc.at[j + 1]], rbuf.at[1], gsem1)
        pltpu.sync_copy(rbuf.at[0], acc.at[dstc.at[j]], add=True)
        pltpu.make_async_copy(tbl.at[srcc.at[j + 1]], rbuf.at[1], gsem1).wait()
        nxt = lax.rem(j + 2, _CPG)
        pltpu.async_copy(tbl.at[srcc.at[nxt]], rbuf.at[0], gsem0)
        pltpu.sync_copy(rbuf.at[1], acc.at[dstc.at[j + 1]], add=True)
        return carry

    lax.fori_loop(0, _CPG // 2, body, 0)
    pltpu.make_async_copy(tbl.at[srcc.at[0]], rbuf.at[0], gsem0).wait()


def _conv_half(tbl, src_hbm, dst_hbm, zrow_hbm, out_half, srcc, dstc, rbuf,
               acc, gsem0, gsem1, s):
    pltpu.sync_copy(zrow_hbm, acc.at[pl.ds(s * _RT, _RT)])
    plsc.subcore_barrier()
    _conv_phase(tbl, src_hbm.at[2 * s], dst_hbm.at[2 * s],
                srcc, dstc, rbuf, acc, gsem0, gsem1)
    _conv_phase(tbl, src_hbm.at[2 * s + 1], dst_hbm.at[2 * s + 1],
                srcc, dstc, rbuf, acc, gsem0, gsem1)
    plsc.subcore_barrier()
    pltpu.sync_copy(acc.at[pl.ds(s * _RT, _RT)], out_half.at[pl.ds(s * _RT, _RT)])


def _sc_conv_body(hs0, hs1, src_hbm, dst_hbm, zrow_hbm, out_hbm, srcc, dstc,
                  rbuf, acc, gsem0, gsem1):
    c = lax.axis_index("c")
    s = lax.axis_index("s")

    @pl.when(c == 0)
    def _():
        _conv_half(hs0, src_hbm, dst_hbm, zrow_hbm, out_hbm.at[0],
                   srcc, dstc, rbuf, acc, gsem0, gsem1, s)

    @pl.when(c == 1)
    def _():
        _conv_half(hs1, src_hbm, dst_hbm, zrow_hbm, out_hbm.at[1],
                   srcc, dstc, rbuf, acc, gsem0, gsem1, s)


@functools.cache
def _sc_kernels():
    mesh = plsc.VectorSubcoreMesh(
        core_axis_name="c", subcore_axis_name="s",
        num_cores=_NC, num_subcores=_NT)
    sc_degree = pl.kernel(
        _sc_degree_body,
        out_type=jax.ShapeDtypeStruct((_NC, _ACC_ROWS, _H), jnp.float32),
        mesh=mesh,
        scratch_types=[
            pltpu.VMEM((_CPG, _CHUNK), jnp.int32),
            pltpu.VMEM((_CHUNK, _H), jnp.float32),
            pltpu.VMEM_SHARED((_ACC_ROWS, _H), jnp.float32),
        ],
    )
    sc_conv = pl.kernel(
        _sc_conv_body,
        out_type=jax.ShapeDtypeStruct((_NC, _ACC_ROWS, _H), jnp.float32),
        mesh=mesh,
        scratch_types=[
            pltpu.VMEM((_CPG, _CHUNK), jnp.int32),
            pltpu.VMEM((_CPG, _CHUNK), jnp.int32),
            pltpu.VMEM((2, _CHUNK, _H), jnp.float32),
            pltpu.VMEM_SHARED((_ACC_ROWS, _H), jnp.float32),
            pltpu.SemaphoreType.DMA,
            pltpu.SemaphoreType.DMA,
        ],
    )
    return sc_degree, sc_conv


def _tc1_body(x_ref, w1_ref, deg_ref, hs0_ref, hs1_ref, dinv_ref):
    h = jnp.dot(x_ref[...], w1_ref[...], preferred_element_type=jnp.float32)
    dinv = lax.rsqrt(deg_ref[0] + deg_ref[1] + 1.0)
    hs = h * dinv
    hs0_ref[...] = hs[:, :_H]
    hs1_ref[...] = hs[:, _H:]
    dinv_ref[...] = jnp.broadcast_to(dinv, (_R, _H))


def _tc2_body(a_ref, hs0_ref, hs1_ref, dinv_ref, b1_ref, wl_ref, bl_ref,
              w2_ref, o0_ref, o1_ref):
    dinv = dinv_ref[...]
    g0 = jnp.maximum(dinv * (a_ref[0] + hs0_ref[...]) + b1_ref[0, :_H], 0.0)
    g1 = jnp.maximum(dinv * (a_ref[1] + hs1_ref[...]) + b1_ref[0, _H:], 0.0)
    g = jnp.concatenate([g0, g1], axis=1)
    z = jnp.maximum(
        jnp.dot(g, wl_ref[...], preferred_element_type=jnp.float32) + bl_ref[...],
        0.0)
    h2 = jnp.dot(z, w2_ref[...], preferred_element_type=jnp.float32)
    o0_ref[...] = h2[:, :_H] * dinv
    o1_ref[...] = h2[:, _H:] * dinv


def _tc3_body(a_ref, hs0_ref, hs1_ref, dinv_ref, b2_ref, out_ref):
    dinv = dinv_ref[...]
    o0 = dinv * (a_ref[0] + hs0_ref[...]) + b2_ref[0, :_H]
    o1 = dinv * (a_ref[1] + hs1_ref[...]) + b2_ref[0, _H:]
    out_ref[...] = jnp.concatenate([o0, o1], axis=1)


_row = pl.BlockSpec((_R, _D), lambda i: (i, 0))
_half = pl.BlockSpec((_R, _H), lambda i: (i, 0))
_whole = pl.BlockSpec((_D, _D), lambda i: (0, 0))
_bias = pl.BlockSpec((1, _D), lambda i: (0, 0))
_acc_spec = pl.BlockSpec((_NC, _R, _H), lambda i: (0, i, 0))

_tc1 = pl.pallas_call(
    _tc1_body,
    grid=(_N // _R,),
    in_specs=[_row, _whole, pl.BlockSpec((_NC, _R, 1), lambda i: (0, i, 0))],
    out_specs=[_half, _half, _half],
    out_shape=[jax.ShapeDtypeStruct((_N, _H), jnp.float32)] * 3,
)

_tc2 = pl.pallas_call(
    _tc2_body,
    grid=(_N // _R,),
    in_specs=[_acc_spec, _half, _half, _half, _bias, _whole, _bias, _whole],
    out_specs=[_half, _half],
    out_shape=[jax.ShapeDtypeStruct((_N, _H), jnp.float32)] * 2,
)

_tc3 = pl.pallas_call(
    _tc3_body,
    grid=(_N // _R,),
    in_specs=[_acc_spec, _half, _half, _half, _bias],
    out_specs=_row,
    out_shape=jax.ShapeDtypeStruct((_N, _D), jnp.float32),
)


def kernel(x, adj, W1, b1, Wl, bl, W2, b2):
    src = adj[0]
    dst = adj[1]
    pad = _EP - _E
    srcp = jnp.concatenate(
        [src, jnp.zeros((pad,), jnp.int32)]).reshape(_GROUPS, _CPG, _CHUNK)
    dstp = jnp.concatenate(
        [dst, jnp.full((pad,), _PAD_ROW, jnp.int32)]).reshape(_GROUPS, _CPG, _CHUNK)
    ones128 = jnp.ones((_CHUNK, _H), jnp.float32)
    zrow = jnp.zeros((_RT, _H), jnp.float32)

    _sc_degree, _sc_conv = _sc_kernels()
    degf = _sc_degree(dstp, ones128, zrow)
    deg = degf[:, :_N, :1]

    hs0, hs1, dinv = _tc1(x, W1, deg)

    acc1 = _sc_conv(hs0, hs1, srcp, dstp, zrow)
    hs20, hs21 = _tc2(acc1[:, :_N, :], hs0, hs1, dinv,
                      b1.reshape(1, _D), Wl, bl.reshape(1, _D), W2)

    acc2 = _sc_conv(hs20, hs21, srcp, dstp, zrow)
    return _tc3(acc2[:, :_N, :], hs20, hs21, dinv, b2.reshape(1, _D))


# speedup vs baseline: 7.4916x; 1.0019x over previous
import functools

import jax
import jax.numpy as jnp
from jax import lax
from jax.experimental import pallas as pl
from jax.experimental.pallas import tpu as pltpu
from jax.experimental.pallas import tpu_sc as plsc

_N = 10000
_D = 256
_H = 128
_E = 160000
_NT = 16
_NC = 2
_CHUNK = 128
_GROUPS = _NC * _NT
_CPG = 40
_SCHUNK = 40
_CPP = 64
_NSLOT = 4
_EP = _GROUPS * _CPG * _CHUNK
_ACC_ROWS = 10240
_RT = _ACC_ROWS // _NT
_PAD_ROW = _N
_R = 1000


def _sc_degree_body(dst_hbm, ones_hbm, zrow_hbm, deg_hbm, dstc, ones_v, acc):
    c = lax.axis_index("c")
    s = lax.axis_index("s")
    pltpu.sync_copy(zrow_hbm, acc.at[pl.ds(s * _RT, _RT)])
    pltpu.sync_copy(ones_hbm, ones_v)
    pltpu.sync_copy(dst_hbm.at[c * _NT + s], dstc)
    plsc.subcore_barrier()

    def body(j, carry):
        pltpu.sync_copy(ones_v, acc.at[dstc.at[j]], add=True)
        return carry

    lax.fori_loop(0, _CPG, body, 0)
    plsc.subcore_barrier()

    @pl.when(c == 0)
    def _():
        pltpu.sync_copy(acc.at[pl.ds(s * _RT, _RT)],
                        deg_hbm.at[0, pl.ds(s * _RT, _RT)])

    @pl.when(c == 1)
    def _():
        pltpu.sync_copy(acc.at[pl.ds(s * _RT, _RT)],
                        deg_hbm.at[1, pl.ds(s * _RT, _RT)])


def _conv_phase(tbl, src_g, dst_g, srcc, dstc, rbuf, acc, gsems, ssems):
    pltpu.sync_copy(src_g, srcc)
    pltpu.sync_copy(dst_g, dstc)
    for k in range(_NSLOT):
        pltpu.async_copy(tbl.at[srcc.at[k]], rbuf.at[k], gsems[k])

    def body(t, carry):
        j = _NSLOT * t
        for k in range(_NSLOT):
            pltpu.make_async_copy(tbl.at[srcc.at[j + k]], rbuf.at[k],
                                  gsems[k]).wait()
            pltpu.async_copy(rbuf.at[k], acc.at[dstc.at[j + k]], ssems[k],
                             add=True)
        for k in range(_NSLOT):
            pltpu.make_async_copy(rbuf.at[k], acc.at[dstc.at[j + k]],
                                  ssems[k]).wait()
            nxt = lax.rem(j + _NSLOT + k, _CPP)
            pltpu.async_copy(tbl.at[srcc.at[nxt]], rbuf.at[k], gsems[k])
        return carry

    lax.fori_loop(0, _CPP // _NSLOT, body, 0)
    for k in range(_NSLOT):
        pltpu.make_async_copy(tbl.at[srcc.at[k]], rbuf.at[k], gsems[k]).wait()


def _conv_half(tbl, src_hbm, dst_hbm, zrow_hbm, out_half, srcc, dstc, rbuf,
               acc, gsems, ssems, s):
    pltpu.sync_copy(zrow_hbm, acc.at[pl.ds(s * _RT, _RT)])
    plsc.subcore_barrier()
    for g in (2 * s, 2 * s + 1):
        for q in (0, 1):
            _conv_phase(tbl, src_hbm.at[g, pl.ds(q * _CPP, _CPP)],
                        dst_hbm.at[g, pl.ds(q * _CPP, _CPP)],
                        srcc, dstc, rbuf, acc, gsems, ssems)
    plsc.subcore_barrier()
    pltpu.sync_copy(acc.at[pl.ds(s * _RT, _RT)], out_half.at[pl.ds(s * _RT, _RT)])


def _sc_conv_body(hs0, hs1, src_hbm, dst_hbm, zrow_hbm, out_hbm, srcc, dstc,
                  rbuf, acc, *sems):
    gsems = sems[:_NSLOT]
    ssems = sems[_NSLOT:]
    c = lax.axis_index("c")
    s = lax.axis_index("s")

    @pl.when(c == 0)
    def _():
        _conv_half(hs0, src_hbm, dst_hbm, zrow_hbm, out_hbm.at[0],
                   srcc, dstc, rbuf, acc, gsems, ssems, s)

    @pl.when(c == 1)
    def _():
        _conv_half(hs1, src_hbm, dst_hbm, zrow_hbm, out_hbm.at[1],
                   srcc, dstc, rbuf, acc, gsems, ssems, s)


@functools.cache
def _sc_kernels():
    mesh = plsc.VectorSubcoreMesh(
        core_axis_name="c", subcore_axis_name="s",
        num_cores=_NC, num_subcores=_NT)
    sc_degree = pl.kernel(
        _sc_degree_body,
        out_type=jax.ShapeDtypeStruct((_NC, _ACC_ROWS, _H), jnp.float32),
        mesh=mesh,
        scratch_types=[
            pltpu.VMEM((_CPG, _CHUNK), jnp.int32),
            pltpu.VMEM((_CHUNK, _H), jnp.float32),
            pltpu.VMEM_SHARED((_ACC_ROWS, _H), jnp.float32),
        ],
    )
    sc_conv = pl.kernel(
        _sc_conv_body,
        out_type=jax.ShapeDtypeStruct((_NC, _ACC_ROWS, _H), jnp.float32),
        mesh=mesh,
        scratch_types=[
            pltpu.VMEM((_CPP, _SCHUNK), jnp.int32),
            pltpu.VMEM((_CPP, _SCHUNK), jnp.int32),
            pltpu.VMEM((_NSLOT, _SCHUNK, _H), jnp.float32),
            pltpu.VMEM_SHARED((_ACC_ROWS, _H), jnp.float32),
        ] + [pltpu.SemaphoreType.DMA] * (2 * _NSLOT),
    )
    return sc_degree, sc_conv


def _tc1_body(x_ref, w1_ref, deg_ref, hs0_ref, hs1_ref, dinv_ref):
    h = jnp.dot(x_ref[...], w1_ref[...], preferred_element_type=jnp.float32)
    dinv = lax.rsqrt(deg_ref[0] + deg_ref[1] + 1.0)
    hs = h * dinv
    hs0_ref[...] = hs[:, :_H]
    hs1_ref[...] = hs[:, _H:]
    dinv_ref[...] = jnp.broadcast_to(dinv, (_R, _H))


def _tc2_body(a_ref, hs0_ref, hs1_ref, dinv_ref, b1_ref, wl_ref, bl_ref,
              w2_ref, o0_ref, o1_ref):
    dinv = dinv_ref[...]
    g0 = jnp.maximum(dinv * (a_ref[0] + hs0_ref[...]) + b1_ref[0, :_H], 0.0)
    g1 = jnp.maximum(dinv * (a_ref[1] + hs1_ref[...]) + b1_ref[0, _H:], 0.0)
    g = jnp.concatenate([g0, g1], axis=1)
    z = jnp.maximum(
        jnp.dot(g, wl_ref[...], preferred_element_type=jnp.float32) + bl_ref[...],
        0.0)
    h2 = jnp.dot(z, w2_ref[...], preferred_element_type=jnp.float32)
    o0_ref[...] = h2[:, :_H] * dinv
    o1_ref[...] = h2[:, _H:] * dinv


def _tc3_body(a_ref, hs0_ref, hs1_ref, dinv_ref, b2_ref, out_ref):
    dinv = dinv_ref[...]
    o0 = dinv * (a_ref[0] + hs0_ref[...]) + b2_ref[0, :_H]
    o1 = dinv * (a_ref[1] + hs1_ref[...]) + b2_ref[0, _H:]
    out_ref[...] = jnp.concatenate([o0, o1], axis=1)


_row = pl.BlockSpec((_R, _D), lambda i: (i, 0))
_half = pl.BlockSpec((_R, _H), lambda i: (i, 0))
_whole = pl.BlockSpec((_D, _D), lambda i: (0, 0))
_bias = pl.BlockSpec((1, _D), lambda i: (0, 0))
_acc_spec = pl.BlockSpec((_NC, _R, _H), lambda i: (0, i, 0))

_tc1 = pl.pallas_call(
    _tc1_body,
    grid=(_N // _R,),
    in_specs=[_row, _whole, pl.BlockSpec((_NC, _R, 1), lambda i: (0, i, 0))],
    out_specs=[_half, _half, _half],
    out_shape=[jax.ShapeDtypeStruct((_N, _H), jnp.float32)] * 3,
)

_tc2 = pl.pallas_call(
    _tc2_body,
    grid=(_N // _R,),
    in_specs=[_acc_spec, _half, _half, _half, _bias, _whole, _bias, _whole],
    out_specs=[_half, _half],
    out_shape=[jax.ShapeDtypeStruct((_N, _H), jnp.float32)] * 2,
)

_tc3 = pl.pallas_call(
    _tc3_body,
    grid=(_N // _R,),
    in_specs=[_acc_spec, _half, _half, _half, _bias],
    out_specs=_row,
    out_shape=jax.ShapeDtypeStruct((_N, _D), jnp.float32),
)


def kernel(x, adj, W1, b1, Wl, bl, W2, b2):
    src = adj[0]
    dst = adj[1]
    pad = _EP - _E
    srcp = jnp.concatenate(
        [src, jnp.zeros((pad,), jnp.int32)]).reshape(_GROUPS, _CPG, _CHUNK)
    dstp = jnp.concatenate(
        [dst, jnp.full((pad,), _PAD_ROW, jnp.int32)]).reshape(_GROUPS, _CPG, _CHUNK)
    ones128 = jnp.ones((_CHUNK, _H), jnp.float32)
    zrow = jnp.zeros((_RT, _H), jnp.float32)

    _sc_degree, _sc_conv = _sc_kernels()
    degf = _sc_degree(dstp, ones128, zrow)
    deg = degf[:, :_N, :1]

    hs0, hs1, dinv = _tc1(x, W1, deg)

    srcp64 = srcp.reshape(_GROUPS, 2 * _CPP, _SCHUNK)
    dstp64 = dstp.reshape(_GROUPS, 2 * _CPP, _SCHUNK)
    acc1 = _sc_conv(hs0, hs1, srcp64, dstp64, zrow)
    hs20, hs21 = _tc2(acc1[:, :_N, :], hs0, hs1, dinv,
                      b1.reshape(1, _D), Wl, bl.reshape(1, _D), W2)

    acc2 = _sc_conv(hs20, hs21, srcp64, dstp64, zrow)
    return _tc3(acc2[:, :_N, :], hs20, hs21, dinv, b2.reshape(1, _D))
